# initial kernel scaffold (unmeasured)
import jax
import jax.numpy as jnp
from jax import lax
from jax.experimental import pallas as pl
from jax.experimental.pallas import tpu as pltpu


def kernel(
    x,
):
    def body(*refs):
        pass

    out_shape = jax.ShapeDtypeStruct(..., jnp.float32)
    return pl.pallas_call(body, out_shape=out_shape)(...)


# baseline (device time: 148961 ns/iter reference)
import jax
import jax.numpy as jnp
from jax import lax
from jax.experimental import pallas as pl
from jax.experimental.pallas import tpu as pltpu

N_DEV = 4


def kernel(x):
    m, n = x.shape

    def body(x_ref, out_ref, comm_ref, send_sems, recv_sems):
        my_pos = lax.axis_index("i")
        left = (my_pos - 1) % N_DEV
        right = (my_pos + 1) % N_DEV

        barrier_sem = pltpu.get_barrier_semaphore()
        for nbr in [left, right]:
            pl.semaphore_signal(
                barrier_sem, inc=1,
                device_id=(nbr,), device_id_type=pl.DeviceIdType.MESH,
            )
        pl.semaphore_wait(barrier_sem, 2)

        out_ref[:, :] = x_ref[:, :]
        comm_ref[0, :, :] = x_ref[:, :]

        for h in range(N_DEV - 1):
            send_slot = h % 2
            recv_slot = (h + 1) % 2
            rdma = pltpu.make_async_remote_copy(
                src_ref=comm_ref.at[send_slot],
                dst_ref=comm_ref.at[recv_slot],
                send_sem=send_sems.at[send_slot],
                recv_sem=recv_sems.at[recv_slot],
                device_id=(right,),
                device_id_type=pl.DeviceIdType.MESH,
            )
            rdma.start()
            rdma.wait()
            out_ref[:, :] += comm_ref[recv_slot, :, :]

    return pl.pallas_call(
        body,
        out_shape=jax.ShapeDtypeStruct((m, n), x.dtype),
        in_specs=[pl.BlockSpec(memory_space=pltpu.VMEM)],
        out_specs=pl.BlockSpec(memory_space=pltpu.VMEM),
        scratch_shapes=[
            pltpu.VMEM((2, m, n), x.dtype),
            pltpu.SemaphoreType.DMA((2,)),
            pltpu.SemaphoreType.DMA((2,)),
        ],
        compiler_params=pltpu.CompilerParams(collective_id=0),
    )(x)


# device time: 47983 ns/iter; 3.1045x vs baseline; 3.1045x over previous
import jax
import jax.numpy as jnp
from jax import lax
from jax.experimental import pallas as pl
from jax.experimental.pallas import tpu as pltpu

N_DEV = 4


def kernel(x):
    m, n = x.shape
    H = m // 2
    Q = H // 2
    E = Q // 2

    def body(x_ref, out_ref, rs1a_ref, rs1b_ref, rs2a_ref, rs2b_ref,
             send_sems, recv_sems):
        i = lax.axis_index("i")
        p1 = i ^ 1
        p2 = 3 - i

        a_keep1 = jnp.where((i == 1) | (i == 2), Q, 0)
        a_send1 = Q - a_keep1
        a_own = a_keep1 + jnp.where(i >= 2, E, 0)
        a_send2 = 2 * a_keep1 + E - a_own
        b_keep1 = jnp.where(i >= 2, Q, 0)
        b_send1 = Q - b_keep1
        b_own = E * i
        b_send2 = b_keep1 + E - (b_own - b_keep1)

        barrier_sem = pltpu.get_barrier_semaphore()
        for nbr in [p1, p2]:
            pl.semaphore_signal(
                barrier_sem, inc=1,
                device_id=(nbr,), device_id_type=pl.DeviceIdType.MESH,
            )
        pl.semaphore_wait(barrier_sem, 2)

        out_ref[:, :] = x_ref[:, :]

        def exchange(src_off, dst_ref_full, dst_off, rows, sem_idx, peer,
                     dst_is_out):
            if dst_is_out:
                dst = dst_ref_full.at[pl.ds(dst_off, rows), :]
            else:
                dst = dst_ref_full
            return pltpu.make_async_remote_copy(
                src_ref=out_ref.at[pl.ds(src_off, rows), :],
                dst_ref=dst,
                send_sem=send_sems.at[sem_idx],
                recv_sem=recv_sems.at[sem_idx],
                device_id=(peer,),
                device_id_type=pl.DeviceIdType.MESH,
            )

        rs1a = exchange(a_send1, rs1a_ref, 0, Q, 0, p1, False)
        rs1b = exchange(H + b_send1, rs1b_ref, 0, Q, 1, p2, False)
        rs1a.start()
        rs1b.start()
        rs1a.wait()
        rs1b.wait()
        out_ref[pl.ds(a_keep1, Q), :] += rs1a_ref[:, :]
        out_ref[pl.ds(H + b_keep1, Q), :] += rs1b_ref[:, :]

        rs2a = exchange(a_send2, rs2a_ref, 0, E, 2, p2, False)
        rs2b = exchange(H + b_send2, rs2b_ref, 0, E, 3, p1, False)
        rs2a.start()
        rs2b.start()
        rs2a.wait()
        rs2b.wait()
        out_ref[pl.ds(a_own, E), :] += rs2a_ref[:, :]
        out_ref[pl.ds(H + b_own, E), :] += rs2b_ref[:, :]

        ag2a = exchange(a_own, out_ref, a_own, E, 4, p2, True)
        ag2b = exchange(H + b_own, out_ref, H + b_own, E, 5, p1, True)
        ag2a.start()
        ag2b.start()
        ag2a.wait()
        ag2b.wait()

        ag1a = exchange(a_keep1, out_ref, a_keep1, Q, 6, p1, True)
        ag1b = exchange(H + b_keep1, out_ref, H + b_keep1, Q, 7, p2, True)
        ag1a.start()
        ag1b.start()
        ag1a.wait()
        ag1b.wait()

    return pl.pallas_call(
        body,
        out_shape=jax.ShapeDtypeStruct((m, n), x.dtype),
        in_specs=[pl.BlockSpec(memory_space=pltpu.VMEM)],
        out_specs=pl.BlockSpec(memory_space=pltpu.VMEM),
        scratch_shapes=[
            pltpu.VMEM((Q, n), x.dtype),
            pltpu.VMEM((Q, n), x.dtype),
            pltpu.VMEM((E, n), x.dtype),
            pltpu.VMEM((E, n), x.dtype),
            pltpu.SemaphoreType.DMA((8,)),
            pltpu.SemaphoreType.DMA((8,)),
        ],
        compiler_params=pltpu.CompilerParams(collective_id=0),
    )(x)


# device time: 46233 ns/iter; 3.2220x vs baseline; 1.0379x over previous
import jax
import jax.numpy as jnp
from jax import lax
from jax.experimental import pallas as pl
from jax.experimental.pallas import tpu as pltpu

N_DEV = 4


def kernel(x):
    m, n = x.shape
    H = m // 2
    Q = H // 2
    E = Q // 2

    def body(x_ref, out_ref,
             rs1a_s2_ref, rs1a_own_ref, rs1b_s2_ref, rs1b_own_ref,
             rs2a_ref, rs2b_ref, send_sems, recv_sems):
        i = lax.axis_index("i")
        b0 = i & 1
        b1 = i >> 1
        p1 = i ^ 1
        p2 = 3 - i

        a_own = Q * (b0 ^ b1) + E * b1
        a_send2 = Q * (b0 ^ b1) + E * (1 - b1)
        a_own_p1 = Q * (1 - (b0 ^ b1)) + E * b1
        a_send2_p1 = Q * (1 - (b0 ^ b1)) + E * (1 - b1)
        b_own = Q * b1 + E * b0
        b_send2 = Q * b1 + E * (1 - b0)
        b_own_p2 = Q * (1 - b1) + E * (1 - b0)
        b_send2_p2 = Q * (1 - b1) + E * b0

        barrier_sem = pltpu.get_barrier_semaphore()
        for nbr in [p1, p2]:
            pl.semaphore_signal(
                barrier_sem, inc=1,
                device_id=(nbr,), device_id_type=pl.DeviceIdType.MESH,
            )
        pl.semaphore_wait(barrier_sem, 2)

        def quarter(src_ref, src_off, dst_ref, dst_off, sem_idx, peer):
            return pltpu.make_async_remote_copy(
                src_ref=src_ref.at[pl.ds(src_off, E), :],
                dst_ref=dst_ref.at[pl.ds(dst_off, E), :],
                send_sem=send_sems.at[sem_idx],
                recv_sem=recv_sems.at[sem_idx],
                device_id=(peer,),
                device_id_type=pl.DeviceIdType.MESH,
            )

        s1a_1 = quarter(x_ref, a_send2_p1, rs1a_s2_ref, 0, 0, p1)
        s1a_2 = quarter(x_ref, a_own_p1, rs1a_own_ref, 0, 1, p1)
        s1b_1 = quarter(x_ref, H + b_send2_p2, rs1b_s2_ref, 0, 2, p2)
        s1b_2 = quarter(x_ref, H + b_own_p2, rs1b_own_ref, 0, 3, p2)
        s1a_1.start()
        s1b_1.start()
        s1a_2.start()
        s1b_2.start()

        s1a_1.wait_recv()
        out_ref[pl.ds(a_send2, E), :] = (
            x_ref[pl.ds(a_send2, E), :] + rs1a_s2_ref[:, :]
        )
        s2a = quarter(out_ref, a_send2, rs2a_ref, 0, 4, p2)
        s2a.start()

        s1b_1.wait_recv()
        out_ref[pl.ds(H + b_send2, E), :] = (
            x_ref[pl.ds(H + b_send2, E), :] + rs1b_s2_ref[:, :]
        )
        s2b = quarter(out_ref, H + b_send2, rs2b_ref, 0, 5, p1)
        s2b.start()

        s1a_2.wait_recv()
        out_ref[pl.ds(a_own, E), :] = (
            x_ref[pl.ds(a_own, E), :] + rs1a_own_ref[:, :]
        )
        s1b_2.wait_recv()
        out_ref[pl.ds(H + b_own, E), :] = (
            x_ref[pl.ds(H + b_own, E), :] + rs1b_own_ref[:, :]
        )

        s2a.wait_recv()
        out_ref[pl.ds(a_own, E), :] += rs2a_ref[:, :]
        ag2a = quarter(out_ref, a_own, out_ref, a_own, 6, p2)
        ag1a_own = quarter(out_ref, a_own, out_ref, a_own, 7, p1)
        ag2a.start()
        ag1a_own.start()

        s2b.wait_recv()
        out_ref[pl.ds(H + b_own, E), :] += rs2b_ref[:, :]
        ag2b = quarter(out_ref, H + b_own, out_ref, H + b_own, 8, p1)
        ag1b_own = quarter(out_ref, H + b_own, out_ref, H + b_own, 9, p2)
        ag2b.start()
        ag1b_own.start()

        ag2a.wait_recv()
        ag1a_oth = quarter(out_ref, a_send2, out_ref, a_send2, 10, p1)
        ag1a_oth.start()

        ag2b.wait_recv()
        ag1b_oth = quarter(out_ref, H + b_send2, out_ref, H + b_send2, 11, p2)
        ag1b_oth.start()

        ag1a_own.wait_recv()
        ag1a_oth.wait_recv()
        ag1b_own.wait_recv()
        ag1b_oth.wait_recv()

        for r in [s1a_1, s1a_2, s1b_1, s1b_2, s2a, s2b,
                  ag2a, ag2b, ag1a_own, ag1b_own, ag1a_oth, ag1b_oth]:
            r.wait_send()

    return pl.pallas_call(
        body,
        out_shape=jax.ShapeDtypeStruct((m, n), x.dtype),
        in_specs=[pl.BlockSpec(memory_space=pltpu.VMEM)],
        out_specs=pl.BlockSpec(memory_space=pltpu.VMEM),
        scratch_shapes=[
            pltpu.VMEM((E, n), x.dtype),
            pltpu.VMEM((E, n), x.dtype),
            pltpu.VMEM((E, n), x.dtype),
            pltpu.VMEM((E, n), x.dtype),
            pltpu.VMEM((E, n), x.dtype),
            pltpu.VMEM((E, n), x.dtype),
            pltpu.SemaphoreType.DMA((12,)),
            pltpu.SemaphoreType.DMA((12,)),
        ],
        compiler_params=pltpu.CompilerParams(collective_id=0),
    )(x)


# device time: 44209 ns/iter; 3.3695x vs baseline; 1.0458x over previous
import jax
import jax.numpy as jnp
from jax import lax
from jax.experimental import pallas as pl
from jax.experimental.pallas import tpu as pltpu

N_DEV = 4


def kernel(x):
    m, n = x.shape
    H = m // 2
    Q = H // 2
    E = Q // 2
    E2 = E // 2

    def body(x_ref, out_ref,
             rs1a_s2_ref, rs1a_own_ref, rs1b_s2_ref, rs1b_own_ref,
             rs2a_ref, rs2b_ref, send_sems, recv_sems):
        i = lax.axis_index("i")
        b0 = i & 1
        b1 = i >> 1
        p1 = i ^ 1
        p2 = 3 - i

        a_own = Q * (b0 ^ b1) + E * b1
        a_send2 = Q * (b0 ^ b1) + E * (1 - b1)
        a_own_p1 = Q * (1 - (b0 ^ b1)) + E * b1
        a_send2_p1 = Q * (1 - (b0 ^ b1)) + E * (1 - b1)
        b_own = Q * b1 + E * b0
        b_send2 = Q * b1 + E * (1 - b0)
        b_own_p2 = Q * (1 - b1) + E * (1 - b0)
        b_send2_p2 = Q * (1 - b1) + E * b0

        barrier_sem = pltpu.get_barrier_semaphore()
        for nbr in [p1, p2]:
            pl.semaphore_signal(
                barrier_sem, inc=1,
                device_id=(nbr,), device_id_type=pl.DeviceIdType.MESH,
            )
        pl.semaphore_wait(barrier_sem, 2)

        def copy(src_ref, src_off, dst_ref, dst_off, rows, sem_idx, peer):
            return pltpu.make_async_remote_copy(
                src_ref=src_ref.at[pl.ds(src_off, rows), :],
                dst_ref=dst_ref.at[pl.ds(dst_off, rows), :],
                send_sem=send_sems.at[sem_idx],
                recv_sem=recv_sems.at[sem_idx],
                device_id=(peer,),
                device_id_type=pl.DeviceIdType.MESH,
            )

        s1a_1 = copy(x_ref, a_send2_p1, rs1a_s2_ref, 0, E, 0, p1)
        s1a_2 = copy(x_ref, a_own_p1, rs1a_own_ref, 0, E, 1, p1)
        s1b_1 = copy(x_ref, H + b_send2_p2, rs1b_s2_ref, 0, E, 2, p2)
        s1b_2 = copy(x_ref, H + b_own_p2, rs1b_own_ref, 0, E, 3, p2)
        s1a_1.start()
        s1b_1.start()
        s1a_2.start()
        s1b_2.start()

        s1a_1.wait_recv()
        out_ref[pl.ds(a_send2, E2), :] = (
            x_ref[pl.ds(a_send2, E2), :] + rs1a_s2_ref[pl.ds(0, E2), :]
        )
        s2a_1 = copy(out_ref, a_send2, rs2a_ref, 0, E2, 4, p2)
        s2a_1.start()
        out_ref[pl.ds(a_send2 + E2, E2), :] = (
            x_ref[pl.ds(a_send2 + E2, E2), :]
            + rs1a_s2_ref[pl.ds(E2, E2), :]
        )
        s2a_2 = copy(out_ref, a_send2 + E2, rs2a_ref, E2, E2, 5, p2)
        s2a_2.start()

        s1b_1.wait_recv()
        out_ref[pl.ds(H + b_send2, E2), :] = (
            x_ref[pl.ds(H + b_send2, E2), :] + rs1b_s2_ref[pl.ds(0, E2), :]
        )
        s2b_1 = copy(out_ref, H + b_send2, rs2b_ref, 0, E2, 6, p1)
        s2b_1.start()
        out_ref[pl.ds(H + b_send2 + E2, E2), :] = (
            x_ref[pl.ds(H + b_send2 + E2, E2), :]
            + rs1b_s2_ref[pl.ds(E2, E2), :]
        )
        s2b_2 = copy(out_ref, H + b_send2 + E2, rs2b_ref, E2, E2, 7, p1)
        s2b_2.start()

        s1a_2.wait_recv()
        out_ref[pl.ds(a_own, E), :] = (
            x_ref[pl.ds(a_own, E), :] + rs1a_own_ref[:, :]
        )
        s1b_2.wait_recv()
        out_ref[pl.ds(H + b_own, E), :] = (
            x_ref[pl.ds(H + b_own, E), :] + rs1b_own_ref[:, :]
        )

        s2a_1.wait_recv()
        out_ref[pl.ds(a_own, E2), :] += rs2a_ref[pl.ds(0, E2), :]
        ag2a_1 = copy(out_ref, a_own, out_ref, a_own, E2, 8, p2)
        ag2a_1.start()
        s2a_2.wait_recv()
        out_ref[pl.ds(a_own + E2, E2), :] += rs2a_ref[pl.ds(E2, E2), :]
        ag2a_2 = copy(out_ref, a_own + E2, out_ref, a_own + E2, E2, 9, p2)
        ag2a_2.start()

        s2b_1.wait_recv()
        out_ref[pl.ds(H + b_own, E2), :] += rs2b_ref[pl.ds(0, E2), :]
        ag2b_1 = copy(out_ref, H + b_own, out_ref, H + b_own, E2, 10, p1)
        ag2b_1.start()
        s2b_2.wait_recv()
        out_ref[pl.ds(H + b_own + E2, E2), :] += rs2b_ref[pl.ds(E2, E2), :]
        ag2b_2 = copy(out_ref, H + b_own + E2, out_ref, H + b_own + E2,
                      E2, 11, p1)
        ag2b_2.start()

        ag1a_own = copy(out_ref, a_own, out_ref, a_own, E, 12, p1)
        ag1a_own.start()
        ag1b_own = copy(out_ref, H + b_own, out_ref, H + b_own, E, 13, p2)
        ag1b_own.start()

        ag2a_1.wait_recv()
        ag2a_2.wait_recv()
        ag1a_oth = copy(out_ref, a_send2, out_ref, a_send2, E, 14, p1)
        ag1a_oth.start()

        ag2b_1.wait_recv()
        ag2b_2.wait_recv()
        ag1b_oth = copy(out_ref, H + b_send2, out_ref, H + b_send2,
                        E, 15, p2)
        ag1b_oth.start()

        ag1a_own.wait_recv()
        ag1a_oth.wait_recv()
        ag1b_own.wait_recv()
        ag1b_oth.wait_recv()

        for r in [s1a_1, s1a_2, s1b_1, s1b_2,
                  s2a_1, s2a_2, s2b_1, s2b_2,
                  ag2a_1, ag2a_2, ag2b_1, ag2b_2,
                  ag1a_own, ag1b_own, ag1a_oth, ag1b_oth]:
            r.wait_send()

    return pl.pallas_call(
        body,
        out_shape=jax.ShapeDtypeStruct((m, n), x.dtype),
        in_specs=[pl.BlockSpec(memory_space=pltpu.VMEM)],
        out_specs=pl.BlockSpec(memory_space=pltpu.VMEM),
        scratch_shapes=[
            pltpu.VMEM((E, n), x.dtype),
            pltpu.VMEM((E, n), x.dtype),
            pltpu.VMEM((E, n), x.dtype),
            pltpu.VMEM((E, n), x.dtype),
            pltpu.VMEM((E, n), x.dtype),
            pltpu.VMEM((E, n), x.dtype),
            pltpu.SemaphoreType.DMA((16,)),
            pltpu.SemaphoreType.DMA((16,)),
        ],
        compiler_params=pltpu.CompilerParams(collective_id=0),
    )(x)
